# baseline (speedup 1.0000x reference)
import jax
import jax.numpy as jnp
from jax import lax
from jax.experimental import pallas as pl
from jax.experimental.pallas import tpu as pltpu
from jax.experimental.pallas import tpu_sc as plsc

EMB = 32
NC = 2
NS = 16
NW = NC * NS
CB = 64


def _gather_call(B, L, idx, table):
    b_per_w = B // NW
    rows_per_ch = CB * L
    n_chunks = b_per_w // CB
    half_rows = (CB // 2) * L
    HB = CB // 2

    LP = (L + 7) // 8 * 8
    MP = 128

    mesh = plsc.VectorSubcoreMesh(
        core_axis_name="c", subcore_axis_name="s", num_cores=NC,
        num_subcores=NS)

    @pl.kernel(
        out_type=jax.ShapeDtypeStruct((B, LP, MP), jnp.float32),
        mesh=mesh,
        compiler_params=pltpu.CompilerParams(use_tc_tiling_on_sc=False),
        scratch_types=[
            pltpu.VMEM((b_per_w, L), jnp.int32),
            pltpu.VMEM((rows_per_ch, EMB), jnp.float32),
            pltpu.SemaphoreType.DMA,
            pltpu.SemaphoreType.DMA,
            pltpu.SemaphoreType.DMA,
        ],
    )
    def k(idx_hbm, table_hbm, out_hbm, idx_v, rows_v, sg, soa, sob):
        wid = lax.axis_index("s") * NC + lax.axis_index("c")
        batch0 = wid * b_per_w

        pltpu.sync_copy(idx_hbm.at[pl.ds(batch0, b_per_w), :], idx_v)

        def fire_gathers(c, h):
            def one(i):
                pltpu.async_copy(
                    table_hbm.at[idx_v.at[c * CB + h * HB + i]],
                    rows_v.at[pl.ds((h * HB + i) * L, L), :],
                    sg,
                )
            pl.loop(0, HB)(one)

        def wait_gathers():
            def one(i):
                pltpu.make_async_copy(
                    table_hbm.at[idx_v.at[0]],
                    rows_v.at[pl.ds(0, L), :],
                    sg,
                ).wait()
            pl.loop(0, HB)(one)

        def fire_outs(c, h, sem):
            def one(bb):
                pltpu.async_copy(
                    rows_v.at[pl.ds((h * HB + bb) * L, L), :],
                    out_hbm.at[batch0 + c * CB + h * HB + bb,
                               pl.ds(0, L), pl.ds(0, EMB)],
                    sem,
                )
            pl.loop(0, HB)(one)

        def wait_outs(sem):
            def one(bb):
                pltpu.make_async_copy(
                    rows_v.at[pl.ds(0, L), :],
                    out_hbm.at[batch0, pl.ds(0, L), pl.ds(0, EMB)],
                    sem,
                ).wait()
            pl.loop(0, HB)(one)

        def chunk(c):
            def drain_prev():
                wait_outs(soa)
                wait_outs(sob)
            pl.when(c > 0)(drain_prev)
            fire_gathers(c, 0)
            wait_gathers()
            fire_outs(c, 0, soa)
            fire_gathers(c, 1)
            wait_gathers()
            fire_outs(c, 1, sob)

        pl.loop(0, n_chunks)(chunk)
        wait_outs(soa)
        wait_outs(sob)

    out_padded = k(idx, table)
    return out_padded[:, :L, :EMB]


def kernel(inputs, table):
    B, L = inputs.shape
    if inputs.dtype != jnp.int32:
        inputs = inputs.astype(jnp.int32)
    return _gather_call(B, L, inputs, table)
